# baseline (speedup 1.0000x reference)
import functools
import math

import jax
import jax.numpy as jnp
from jax import lax
from jax.experimental import pallas as pl
from jax.experimental.pallas import tpu as pltpu
from jax.experimental.pallas import tpu_sc as plsc

VOC_SIZE = 1000000
SIZE = 64
MAX_LEN = 200
B = 4096
L = 200
DIVS = 10000.0
SCALE = math.sqrt(SIZE)
BBLK = 128
PITCH = 129


def _pos_enc_table():
    pos = jnp.arange(MAX_LEN, dtype=jnp.float32)[:, None]
    loc_even = jnp.arange(0, SIZE, 2, dtype=jnp.float32)[None, :]
    even_vals = jnp.sin(pos / (DIVS ** (2.0 * loc_even / SIZE)))
    odd_vals = jnp.cos(pos / (DIVS ** (2.0 * (loc_even + 1.0) / SIZE)))
    out = jnp.zeros((MAX_LEN, SIZE), dtype=jnp.float32)
    out = out.at[:, 0::2].set(even_vals)
    out = out.at[:, 1::2].set(odd_vals)
    return out.reshape(-1)


def _make_sc_kernel():
    info = plsc.get_sparse_core_info()
    nc, ns, lanes = info.num_cores, info.num_subcores, info.num_lanes
    nw = nc * ns
    assert B // BBLK == nw
    mesh = plsc.VectorSubcoreMesh(
        core_axis_name="c", subcore_axis_name="s",
        num_cores=nc, num_subcores=ns)

    @functools.partial(
        pl.kernel,
        out_type=jax.ShapeDtypeStruct((L, SIZE, B), jnp.float32),
        mesh=mesh,
        compiler_params=pltpu.CompilerParams(needs_layout_passes=False),
        scratch_types=[
            pltpu.VMEM((L, BBLK), jnp.int32),
            pltpu.VMEM((2, BBLK, 128), jnp.float32),
            pltpu.VMEM((2, SIZE, PITCH), jnp.float32),
            pltpu.VMEM((MAX_LEN * SIZE,), jnp.float32),
            pltpu.SemaphoreType.DMA,
            pltpu.SemaphoreType.DMA,
        ],
    )
    def k(idxT_hbm, t128_hbm, pe_hbm, out_hbm, idx_v, rows_v, outb_v, pe_v,
          gsem, osem):
        wid = lax.axis_index("s") * nc + lax.axis_index("c")
        b0 = wid * BBLK
        pltpu.sync_copy(pe_hbm, pe_v)
        pltpu.sync_copy(idxT_hbm.at[:, pl.ds(b0, BBLK)], idx_v)
        lane_iota = jax.lax.iota(jnp.int32, lanes)

        def gather_start(l, buf):
            return pltpu.async_copy(
                t128_hbm.at[idx_v.at[l]], rows_v.at[buf], gsem)

        def compute(l, buf, obuf):
            rows = rows_v.at[buf]
            outb = outb_v.at[obuf]
            pe_vecs = [
                pe_v[pl.ds(l * SIZE + c * lanes, lanes)]
                for c in range(SIZE // lanes)
            ]
            row_vecs = [lane_iota + c * lanes for c in range(SIZE // lanes)]

            @plsc.parallel_loop(0, BBLK, unroll=8)
            def row_body(b):
                colv = lane_iota * 0 + b
                for c in range(SIZE // lanes):
                    vals = rows[b, pl.ds(c * lanes, lanes)] + pe_vecs[c]
                    plsc.store_scatter(outb, [row_vecs[c], colv], vals)

        def out_start(l, obuf):
            return pltpu.async_copy(
                outb_v.at[obuf, :, pl.ds(0, BBLK)],
                out_hbm.at[l, :, pl.ds(b0, BBLK)], osem)

        def out_wait(l, obuf):
            pltpu.make_async_copy(
                outb_v.at[obuf, :, pl.ds(0, BBLK)],
                out_hbm.at[l, :, pl.ds(b0, BBLK)], osem).wait()

        NB = 2
        for p in range(NB - 1):
            gather_start(p, p)

        def step(l, buf, obuf):
            @pl.when(l < L - (NB - 1))
            def _():
                gather_start(l + NB - 1, (buf + NB - 1) % NB)

            pltpu.make_async_copy(
                t128_hbm.at[idx_v.at[l]], rows_v.at[buf], gsem).wait()

            @pl.when(l >= 2)
            def _():
                out_wait(l - 2, obuf)

            compute(l, buf, obuf)
            out_start(l, obuf)

        def s_body(j, carry):
            for ph in range(NB):
                step(NB * j + ph, ph, ph % 2)
            return carry

        lax.fori_loop(0, L // NB, s_body, 0)
        out_wait(L - 2, 0)
        out_wait(L - 1, 1)

    return k


def kernel(enc_out, table):
    idxT = enc_out.T.astype(jnp.int32)
    t128 = jnp.pad(table * SCALE, ((0, 0), (0, 64)))
    pe = _pos_enc_table()
    k = _make_sc_kernel()
    out_phys = k(idxT, t128, pe)
    return jnp.transpose(out_phys, (2, 0, 1))


# speedup vs baseline: 1.0130x; 1.0130x over previous
import functools
import math

import jax
import jax.numpy as jnp
from jax import lax
from jax.experimental import pallas as pl
from jax.experimental.pallas import tpu as pltpu
from jax.experimental.pallas import tpu_sc as plsc

VOC_SIZE = 1000000
SIZE = 64
MAX_LEN = 200
B = 4096
L = 200
DIVS = 10000.0
SCALE = math.sqrt(SIZE)
BBLK = 128
PITCH = 128


def _pos_enc_table():
    pos = jnp.arange(MAX_LEN, dtype=jnp.float32)[:, None]
    loc_even = jnp.arange(0, SIZE, 2, dtype=jnp.float32)[None, :]
    even_vals = jnp.sin(pos / (DIVS ** (2.0 * loc_even / SIZE)))
    odd_vals = jnp.cos(pos / (DIVS ** (2.0 * (loc_even + 1.0) / SIZE)))
    out = jnp.zeros((MAX_LEN, SIZE), dtype=jnp.float32)
    out = out.at[:, 0::2].set(even_vals)
    out = out.at[:, 1::2].set(odd_vals)
    return out.reshape(-1)


def _make_sc_kernel():
    info = plsc.get_sparse_core_info()
    nc, ns, lanes = info.num_cores, info.num_subcores, info.num_lanes
    nw = nc * ns
    assert B // BBLK == nw
    mesh = plsc.VectorSubcoreMesh(
        core_axis_name="c", subcore_axis_name="s",
        num_cores=nc, num_subcores=ns)

    @functools.partial(
        pl.kernel,
        out_type=jax.ShapeDtypeStruct((L, SIZE, B), jnp.float32),
        mesh=mesh,
        compiler_params=pltpu.CompilerParams(needs_layout_passes=False),
        scratch_types=[
            pltpu.VMEM((L, BBLK), jnp.int32),
            pltpu.VMEM((2, BBLK, 128), jnp.float32),
            pltpu.VMEM((2, SIZE, PITCH), jnp.float32),
            pltpu.VMEM((MAX_LEN * SIZE,), jnp.float32),
            pltpu.SemaphoreType.DMA,
            pltpu.SemaphoreType.DMA,
        ],
    )
    def k(idxT_hbm, t128_hbm, pe_hbm, out_hbm, idx_v, rows_v, outb_v, pe_v,
          gsem, osem):
        wid = lax.axis_index("s") * nc + lax.axis_index("c")
        b0 = wid * BBLK
        pltpu.sync_copy(pe_hbm, pe_v)
        pltpu.sync_copy(idxT_hbm.at[:, pl.ds(b0, BBLK)], idx_v)
        lane_iota = jax.lax.iota(jnp.int32, lanes)

        def gather_start(l, buf):
            return pltpu.async_copy(
                t128_hbm.at[idx_v.at[l]], rows_v.at[buf], gsem)

        def compute(l, buf, obuf):
            rows = rows_v.at[buf]
            outb = outb_v.at[obuf]
            @plsc.parallel_loop(0, SIZE, unroll=4)
            def d_body(d):
                dvec = lane_iota * 0 + d
                pes = plsc.load_gather(pe_v, [l * SIZE + dvec])
                for j in range(BBLK // lanes):
                    bvec = lane_iota + j * lanes
                    vals = plsc.load_gather(rows, [bvec, dvec]) + pes
                    outb[d, pl.ds(j * lanes, lanes)] = vals

        def out_start(l, obuf):
            return pltpu.async_copy(
                outb_v.at[obuf, :, pl.ds(0, BBLK)],
                out_hbm.at[l, :, pl.ds(b0, BBLK)], osem)

        def out_wait(l, obuf):
            pltpu.make_async_copy(
                outb_v.at[obuf, :, pl.ds(0, BBLK)],
                out_hbm.at[l, :, pl.ds(b0, BBLK)], osem).wait()

        NB = 2
        for p in range(NB - 1):
            gather_start(p, p)

        def step(l, buf, obuf):
            @pl.when(l < L - (NB - 1))
            def _():
                gather_start(l + NB - 1, (buf + NB - 1) % NB)

            pltpu.make_async_copy(
                t128_hbm.at[idx_v.at[l]], rows_v.at[buf], gsem).wait()

            @pl.when(l >= 2)
            def _():
                out_wait(l - 2, obuf)

            compute(l, buf, obuf)
            out_start(l, obuf)

        def s_body(j, carry):
            for ph in range(NB):
                step(NB * j + ph, ph, ph % 2)
            return carry

        lax.fori_loop(0, L // NB, s_body, 0)
        out_wait(L - 2, 0)
        out_wait(L - 1, 1)

    return k


def kernel(enc_out, table):
    idxT = enc_out.T.astype(jnp.int32)
    t128 = jnp.pad(table * SCALE, ((0, 0), (0, 64)))
    pe = _pos_enc_table()
    k = _make_sc_kernel()
    out_phys = k(idxT, t128, pe)
    return jnp.transpose(out_phys, (2, 0, 1))
